# Initial kernel scaffold; baseline (speedup 1.0000x reference)
import jax
import jax.numpy as jnp
from jax.experimental import pallas as pl


def kernel(x, edge_index, W):
    raise NotImplementedError("write your pallas kernel here")


# speedup vs baseline: 29.8907x; 29.8907x over previous
import functools

import jax
import jax.numpy as jnp
from jax import lax
from jax.experimental import pallas as pl
from jax.experimental.pallas import tpu as pltpu
from jax.experimental.pallas import tpu_sc as plsc

N = 10000
D = 128
NC = 2
NS = 16
NW = NC * NS

N_PAD = 10240
ROWS_PER_TILE = N_PAD // NS
CHUNK = 128

_sc_mesh = plsc.VectorSubcoreMesh(core_axis_name="c", subcore_axis_name="s")


def _make_deg_kernel(chunks_per_tile):
    @functools.partial(
        pl.kernel,
        out_type=jax.ShapeDtypeStruct((NC, N_PAD), jnp.float32),
        mesh=_sc_mesh,
        scratch_types=[
            pltpu.VMEM((chunks_per_tile, CHUNK), jnp.int32),
            pltpu.VMEM((CHUNK,), jnp.float32),
            pltpu.VMEM_SHARED((N_PAD,), jnp.float32),
        ],
    )
    def deg_kernel(col_hbm, zeros_hbm, out_hbm, colv, onesv, deg):
        cid = lax.axis_index("c")
        sid = lax.axis_index("s")
        wid = sid * NC + cid
        for i in range(CHUNK // 16):
            onesv[pl.ds(i * 16, 16)] = jnp.full((16,), 1.0, jnp.float32)
        pltpu.sync_copy(zeros_hbm, deg.at[pl.ds(sid * ROWS_PER_TILE, ROWS_PER_TILE)])
        pltpu.sync_copy(col_hbm.at[wid], colv)
        plsc.subcore_barrier()

        def body(j, carry):
            pltpu.sync_copy(onesv, deg.at[colv.at[j]], add=True)
            return carry

        lax.fori_loop(0, chunks_per_tile, body, 0)
        plsc.subcore_barrier()
        pltpu.sync_copy(
            deg.at[pl.ds(sid * ROWS_PER_TILE, ROWS_PER_TILE)],
            out_hbm.at[cid, pl.ds(sid * ROWS_PER_TILE, ROWS_PER_TILE)],
        )

    return deg_kernel


def _make_spmm_kernel(chunks_per_tile):
    @functools.partial(
        pl.kernel,
        out_type=jax.ShapeDtypeStruct((NC, N_PAD, D), jnp.float32),
        mesh=_sc_mesh,
        scratch_types=[
            pltpu.VMEM((chunks_per_tile, CHUNK), jnp.int32),
            pltpu.VMEM((chunks_per_tile, CHUNK), jnp.int32),
            pltpu.VMEM((CHUNK, D), jnp.float32),
            pltpu.VMEM_SHARED((N_PAD, D), jnp.float32),
        ],
    )
    def spmm_kernel(g_hbm, row_hbm, col_hbm, zeros_hbm, out_hbm, rowv, colv, buf, acc):
        cid = lax.axis_index("c")
        sid = lax.axis_index("s")
        wid = sid * NC + cid
        pltpu.sync_copy(zeros_hbm, acc.at[pl.ds(sid * ROWS_PER_TILE, ROWS_PER_TILE)])
        pltpu.sync_copy(row_hbm.at[wid], rowv)
        pltpu.sync_copy(col_hbm.at[wid], colv)
        plsc.subcore_barrier()

        def body(j, carry):
            pltpu.sync_copy(g_hbm.at[colv.at[j]], buf)
            pltpu.sync_copy(buf, acc.at[rowv.at[j]], add=True)
            return carry

        lax.fori_loop(0, chunks_per_tile, body, 0)
        plsc.subcore_barrier()
        pltpu.sync_copy(
            acc.at[pl.ds(sid * ROWS_PER_TILE, ROWS_PER_TILE)],
            out_hbm.at[cid, pl.ds(sid * ROWS_PER_TILE, ROWS_PER_TILE)],
        )

    return spmm_kernel


_BLK = 1024


def _lin_body(x_ref, w_ref, deg_ref, g_ref, dinv_ref):
    dsum = deg_ref[0] + deg_ref[1] + 1.0
    dinv = lax.rsqrt(dsum)
    h = lax.dot_general(
        x_ref[...], w_ref[...],
        dimension_numbers=(((1,), (1,)), ((), ())),
        preferred_element_type=jnp.float32,
    )
    g_ref[...] = h * dinv
    dinv_ref[...] = dinv


def _lin_scale(x_pad, w, deg2):
    grid = N_PAD // _BLK
    return pl.pallas_call(
        _lin_body,
        grid=(grid,),
        in_specs=[
            pl.BlockSpec((_BLK, D), lambda i: (i, 0)),
            pl.BlockSpec((D, D), lambda i: (0, 0)),
            pl.BlockSpec((NC, _BLK, 1), lambda i: (0, i, 0)),
        ],
        out_specs=[
            pl.BlockSpec((_BLK, D), lambda i: (i, 0)),
            pl.BlockSpec((_BLK, 1), lambda i: (i, 0)),
        ],
        out_shape=[
            jax.ShapeDtypeStruct((N_PAD, D), jnp.float32),
            jax.ShapeDtypeStruct((N_PAD, 1), jnp.float32),
        ],
    )(x_pad, w, deg2)


def _combine_body(acc_ref, g_ref, dinv_ref, out_ref):
    out_ref[...] = dinv_ref[...] * (acc_ref[0] + acc_ref[1] + g_ref[...])


def _combine(acc, g, dinv):
    grid = N_PAD // _BLK
    return pl.pallas_call(
        _combine_body,
        grid=(grid,),
        in_specs=[
            pl.BlockSpec((NC, _BLK, D), lambda i: (0, i, 0)),
            pl.BlockSpec((_BLK, D), lambda i: (i, 0)),
            pl.BlockSpec((_BLK, 1), lambda i: (i, 0)),
        ],
        out_specs=pl.BlockSpec((_BLK, D), lambda i: (i, 0)),
        out_shape=jax.ShapeDtypeStruct((N_PAD, D), jnp.float32),
    )(acc, g, dinv)


def kernel(x, edge_index, W):
    n, d = x.shape
    e = edge_index.shape[1]
    chunks_total = -(-e // (NW * CHUNK)) * NW
    chunks_per_tile = chunks_total // NW
    e_pad = chunks_total * CHUNK

    pad = e_pad - e
    spread = jnp.arange(pad, dtype=jnp.int32) % (N_PAD - n) + n
    row = jnp.concatenate([edge_index[0], spread])
    col = jnp.concatenate([edge_index[1], spread])
    row3 = row.reshape(NW, chunks_per_tile, CHUNK)
    col3 = col.reshape(NW, chunks_per_tile, CHUNK)

    x_pad = jnp.zeros((N_PAD, d), x.dtype).at[:n].set(x)
    zeros1 = jnp.zeros((ROWS_PER_TILE,), jnp.float32)
    zeros2 = jnp.zeros((ROWS_PER_TILE, d), jnp.float32)

    deg2 = _make_deg_kernel(chunks_per_tile)(col3, zeros1)
    g, dinv = _lin_scale(x_pad, W, deg2.reshape(NC, N_PAD, 1))
    acc = _make_spmm_kernel(chunks_per_tile)(g, row3, col3, zeros2)
    out = _combine(acc, g, dinv)
    return out[:n]


# speedup vs baseline: 41.6060x; 1.3919x over previous
import functools

import jax
import jax.numpy as jnp
from jax import lax
from jax.experimental import pallas as pl
from jax.experimental.pallas import tpu as pltpu
from jax.experimental.pallas import tpu_sc as plsc

N = 10000
D = 128
NC = 2
NS = 16
NW = NC * NS

N_PAD = 10240
ROWS_PER_TILE = N_PAD // NS
CHUNK = 128
NB = 2
PACK = 14

_sc_mesh = plsc.VectorSubcoreMesh(core_axis_name="c", subcore_axis_name="s")


def _make_deg_kernel(chunks_per_tile):
    @functools.partial(
        pl.kernel,
        out_type=jax.ShapeDtypeStruct((NC, N_PAD), jnp.float32),
        mesh=_sc_mesh,
        scratch_types=[
            pltpu.VMEM((chunks_per_tile, CHUNK), jnp.int32),
            pltpu.VMEM((CHUNK,), jnp.float32),
            pltpu.VMEM_SHARED((N_PAD,), jnp.float32),
        ],
    )
    def deg_kernel(col_hbm, zeros_hbm, out_hbm, colv, onesv, deg):
        cid = lax.axis_index("c")
        sid = lax.axis_index("s")
        wid = sid * NC + cid
        for i in range(CHUNK // 16):
            onesv[pl.ds(i * 16, 16)] = jnp.full((16,), 1.0, jnp.float32)
        pltpu.sync_copy(zeros_hbm, deg.at[pl.ds(sid * ROWS_PER_TILE, ROWS_PER_TILE)])
        pltpu.sync_copy(col_hbm.at[wid], colv)
        plsc.subcore_barrier()

        def body(j, carry):
            pltpu.sync_copy(onesv, deg.at[colv.at[j]], add=True)
            return carry

        lax.fori_loop(0, chunks_per_tile, body, 0)
        plsc.subcore_barrier()
        pltpu.sync_copy(
            deg.at[pl.ds(sid * ROWS_PER_TILE, ROWS_PER_TILE)],
            out_hbm.at[cid, pl.ds(sid * ROWS_PER_TILE, ROWS_PER_TILE)],
        )

    return deg_kernel


def _make_spmm_kernel(chunks_per_tile):
    assert chunks_per_tile % NB == 0
    ngroups = chunks_per_tile // NB

    @functools.partial(
        pl.kernel,
        out_type=jax.ShapeDtypeStruct((NC, N_PAD, D), jnp.float32),
        mesh=_sc_mesh,
        scratch_types=[
            pltpu.VMEM((chunks_per_tile, CHUNK), jnp.int32),
            [pltpu.VMEM((1, CHUNK), jnp.int32) for _ in range(NB)],
            [pltpu.VMEM((1, CHUNK), jnp.int32) for _ in range(NB)],
            [pltpu.VMEM((CHUNK, D), jnp.float32) for _ in range(NB)],
            [pltpu.SemaphoreType.DMA for _ in range(NB)],
            pltpu.VMEM_SHARED((N_PAD, D), jnp.float32),
        ],
    )
    def spmm_kernel(g_hbm, packed_hbm, zeros_hbm, out_hbm,
                    packv, rowbs, colbs, bufs, sems, acc):
        cid = lax.axis_index("c")
        sid = lax.axis_index("s")
        wid = sid * NC + cid
        pltpu.sync_copy(zeros_hbm, acc.at[pl.ds(sid * ROWS_PER_TILE, ROWS_PER_TILE)])
        pltpu.sync_copy(packed_hbm.at[wid], packv)
        plsc.subcore_barrier()

        def decode(j, b):
            for i in range(CHUNK // 16):
                v = packv[j, pl.ds(i * 16, 16)]
                rowbs[b][0, pl.ds(i * 16, 16)] = lax.shift_right_logical(v, PACK)
                colbs[b][0, pl.ds(i * 16, 16)] = lax.bitwise_and(
                    v, jnp.int32((1 << PACK) - 1))

        def gather(b):
            pltpu.async_copy(g_hbm.at[colbs[b].at[0]], bufs[b], sems[b])

        def wait_gather(b):
            pltpu.make_async_copy(g_hbm.at[colbs[b].at[0]], bufs[b],
                                  sems[b]).wait()

        for b in range(NB):
            decode(b, b)
            gather(b)

        def body(jj, carry):
            j0 = jj * NB
            for b in range(NB):
                wait_gather(b)
                pltpu.sync_copy(bufs[b], acc.at[rowbs[b].at[0]], add=True)
                decode(j0 + b + NB, b)
                gather(b)
            return carry

        lax.fori_loop(0, ngroups - 1, body, 0)
        for b in range(NB):
            wait_gather(b)
            pltpu.sync_copy(bufs[b], acc.at[rowbs[b].at[0]], add=True)
        plsc.subcore_barrier()
        pltpu.sync_copy(
            acc.at[pl.ds(sid * ROWS_PER_TILE, ROWS_PER_TILE)],
            out_hbm.at[cid, pl.ds(sid * ROWS_PER_TILE, ROWS_PER_TILE)],
        )

    return spmm_kernel


_BLK = 1024


def _lin_body(x_ref, w_ref, deg_ref, g_ref, dinv_ref):
    dsum = deg_ref[0] + deg_ref[1] + 1.0
    dinv = lax.rsqrt(dsum)
    h = lax.dot_general(
        x_ref[...], w_ref[...],
        dimension_numbers=(((1,), (1,)), ((), ())),
        preferred_element_type=jnp.float32,
    )
    g_ref[...] = h * dinv
    dinv_ref[...] = dinv


def _lin_scale(x_pad, w, deg2):
    grid = N_PAD // _BLK
    return pl.pallas_call(
        _lin_body,
        grid=(grid,),
        in_specs=[
            pl.BlockSpec((_BLK, D), lambda i: (i, 0)),
            pl.BlockSpec((D, D), lambda i: (0, 0)),
            pl.BlockSpec((NC, _BLK, 1), lambda i: (0, i, 0)),
        ],
        out_specs=[
            pl.BlockSpec((_BLK, D), lambda i: (i, 0)),
            pl.BlockSpec((_BLK, 1), lambda i: (i, 0)),
        ],
        out_shape=[
            jax.ShapeDtypeStruct((N_PAD, D), jnp.float32),
            jax.ShapeDtypeStruct((N_PAD, 1), jnp.float32),
        ],
    )(x_pad, w, deg2)


def _combine_body(acc_ref, g_ref, dinv_ref, out_ref):
    out_ref[...] = dinv_ref[...] * (acc_ref[0] + acc_ref[1] + g_ref[...])


def _combine(acc, g, dinv):
    blk = 1000
    grid = N // blk
    return pl.pallas_call(
        _combine_body,
        grid=(grid,),
        in_specs=[
            pl.BlockSpec((NC, blk, D), lambda i: (0, i, 0)),
            pl.BlockSpec((blk, D), lambda i: (i, 0)),
            pl.BlockSpec((blk, 1), lambda i: (i, 0)),
        ],
        out_specs=pl.BlockSpec((blk, D), lambda i: (i, 0)),
        out_shape=jax.ShapeDtypeStruct((N, D), jnp.float32),
    )(acc, g, dinv)


def kernel(x, edge_index, W):
    n, d = x.shape
    e = edge_index.shape[1]
    chunks_per_tile = -(-e // (NW * CHUNK))
    chunks_per_tile = -(-chunks_per_tile // NB) * NB
    e_pad = chunks_per_tile * NW * CHUNK

    pad = e_pad - e
    spread = jnp.arange(pad, dtype=jnp.int32) % (N_PAD - n) + n
    row = jnp.concatenate([edge_index[0], spread])
    col = jnp.concatenate([edge_index[1], spread])
    packed3 = ((row << PACK) | col).reshape(NW, chunks_per_tile, CHUNK)
    col3 = col.reshape(NW, chunks_per_tile, CHUNK)

    x_pad = jnp.zeros((N_PAD, d), x.dtype).at[:n].set(x)
    zeros1 = jnp.zeros((ROWS_PER_TILE,), jnp.float32)
    zeros2 = jnp.zeros((ROWS_PER_TILE, d), jnp.float32)

    deg2 = _make_deg_kernel(chunks_per_tile)(col3, zeros1)
    g, dinv = _lin_scale(x_pad, W, deg2.reshape(NC, N_PAD, 1))
    acc = _make_spmm_kernel(chunks_per_tile)(g, packed3, zeros2)
    return _combine(acc, g, dinv)
